# baseline (speedup 1.0000x reference)
import functools

import numpy as np
import jax
import jax.numpy as jnp
from jax import lax
from jax.experimental import pallas as pl
from jax.experimental.pallas import tpu as pltpu
from jax.experimental.pallas import tpu_sc as plsc

VOCAB = 100000
EMB = 128
N_NEG = 128
BATCH = 4096
CTX = 20

NC, NS = 2, 16
NW = NC * NS
BPW = BATCH // NW
EMBW = EMB // 2
NCH = EMBW // 16

_ILV = plsc.PackFormat.INTERLEAVED


def _neg_wids():
    wids = jax.random.randint(jax.random.key(1234), (BATCH, N_NEG), 0, VOCAB - 1)
    return wids.astype(jnp.int32)


def _pack_table(t):
    tb = t.astype(jnp.bfloat16)
    return lax.bitcast_convert_type(tb.reshape(VOCAB + 1, EMBW, 2), jnp.int32)


def _sc_scores(tgt, ctx, neg, i_emb, o_emb):
    mesh = plsc.VectorSubcoreMesh(
        core_axis_name="c", subcore_axis_name="s", num_cores=NC, num_subcores=NS)

    @functools.partial(
        pl.kernel,
        out_type=(jax.ShapeDtypeStruct((BATCH, 16), jnp.float32),
                  jax.ShapeDtypeStruct((BATCH, N_NEG), jnp.float32)),
        mesh=mesh,
        compiler_params=pltpu.CompilerParams(
            needs_layout_passes=False, use_tc_tiling_on_sc=False),
        scratch_types=[
            pltpu.VMEM((BPW, CTX), jnp.int32),
            pltpu.VMEM((BPW,), jnp.int32),
            pltpu.VMEM((BPW, N_NEG), jnp.int32),
            pltpu.VMEM((BPW, EMBW), jnp.int32),
            pltpu.VMEM((CTX, EMBW), jnp.int32),
            pltpu.VMEM((CTX, EMBW), jnp.int32),
            pltpu.VMEM((N_NEG, EMBW), jnp.int32),
            pltpu.VMEM((N_NEG, EMBW), jnp.int32),
            pltpu.VMEM((BPW, 16), jnp.float32),
            pltpu.VMEM((N_NEG,), jnp.float32),
            pltpu.VMEM((N_NEG,), jnp.float32),
            pltpu.VMEM((N_NEG, 17), jnp.float32),
            pltpu.SemaphoreType.DMA,
            pltpu.SemaphoreType.DMA,
            pltpu.SemaphoreType.DMA,
            pltpu.SemaphoreType.DMA,
            pltpu.SemaphoreType.DMA,
            pltpu.SemaphoreType.DMA,
            pltpu.SemaphoreType.DMA,
        ],
    )
    def k(tgt_h, ctx_h, neg_h, iemb_h, oemb_h, pos_o, negs_o,
          ctx_idx, tgt_idx, neg_idx, tgt_rows, ctx_rows0, ctx_rows1,
          neg_rows0, neg_rows1, pos_v, negs_v0, negs_v1, part_buf,
          sem, sem_c0, sem_c1, sem_n0, sem_n1, sem_o0, sem_o1):
        wid = lax.axis_index("s") * NC + lax.axis_index("c")
        base = wid * BPW
        pltpu.sync_copy(ctx_h.at[pl.ds(base, BPW)], ctx_idx)
        pltpu.sync_copy(tgt_h.at[pl.ds(base, BPW)], tgt_idx)
        pltpu.sync_copy(neg_h.at[pl.ds(base, BPW)], neg_idx)

        sems_c = (sem_c0, sem_c1)
        sems_n = (sem_n0, sem_n1)
        sems_o = (sem_o0, sem_o1)
        ctx_bufs = (ctx_rows0, ctx_rows1)
        neg_bufs = (neg_rows0, neg_rows1)
        out_bufs = (negs_v0, negs_v1)
        lane = lax.iota(jnp.int32, 16)

        def start_ctx(b, buf):
            bb = jnp.minimum(b, BPW - 1)
            pltpu.async_copy(iemb_h.at[ctx_idx.at[bb]], ctx_bufs[buf], sems_c[buf])

        def start_neg(b, buf):
            bb = jnp.minimum(b, BPW - 1)
            pltpu.async_copy(oemb_h.at[neg_idx.at[bb]], neg_bufs[buf], sems_n[buf])

        def wait_ctx(buf):
            pltpu.make_async_copy(
                iemb_h.at[ctx_idx.at[0]], ctx_bufs[buf], sems_c[buf]).wait()

        def wait_neg(buf):
            pltpu.make_async_copy(
                oemb_h.at[neg_idx.at[0]], neg_bufs[buf], sems_n[buf]).wait()

        def row_chunk_bf16(ref, r, j):
            return plsc.bitcast(ref[r, pl.ds(j * 16, 16)], jnp.bfloat16)

        def process(b, buf):
            wait_ctx(buf)
            avg_bf = []
            for j in range(NCH):
                es, os_ = [], []
                for c in range(CTX):
                    e, o = plsc.unpack(row_chunk_bf16(ctx_bufs[buf], c, j),
                                       format=_ILV)
                    es.append(e)
                    os_.append(o)
                for terms in (es, os_):
                    while len(terms) > 1:
                        nxt = [terms[i] + terms[i + 1]
                               for i in range(0, len(terms) - 1, 2)]
                        if len(terms) % 2:
                            nxt.append(terms[-1])
                        terms[:] = nxt
                avg_bf.append(plsc.pack(es[0] / jnp.float32(CTX),
                                        os_[0] / jnp.float32(CTX), format=_ILV))
            start_ctx(b + 2, buf)

            def row_dot_partial(ref, r):
                prods = [row_chunk_bf16(ref, r, j) * avg_bf[j]
                         for j in range(NCH)]
                q = (prods[0] + prods[1]) + (prods[2] + prods[3])
                e, o = plsc.unpack(q, format=_ILV)
                return e + o

            pos_v[b, :] = row_dot_partial(tgt_rows, b)

            wait_neg(buf)
            @pl.when(b >= 2)
            def _():
                pltpu.make_async_copy(
                    out_bufs[buf], negs_o.at[base], sems_o[buf]).wait()

            def neg_group(g):
                base_r = g * 16
                for r in range(16):
                    part_buf[base_r + r, pl.ds(0, 16)] = row_dot_partial(
                        neg_bufs[buf], base_r + r)
                row_idx = base_r + lane
                cols = [plsc.load_gather(
                            part_buf, [row_idx, jnp.full((16,), j, jnp.int32)])
                        for j in range(16)]
                while len(cols) > 1:
                    cols = [cols[i] + cols[i + 1] for i in range(0, len(cols), 2)]
                out_bufs[buf][pl.ds(base_r, 16)] = cols[0]

            plsc.parallel_loop(0, N_NEG // 16, 1)(neg_group)
            pltpu.async_copy(out_bufs[buf], negs_o.at[base + b], sems_o[buf])
            start_neg(b + 2, buf)

        start_ctx(jnp.int32(0), 0)
        start_neg(jnp.int32(0), 0)
        start_ctx(jnp.int32(1), 1)
        start_neg(jnp.int32(1), 1)
        pltpu.async_copy(oemb_h.at[tgt_idx], tgt_rows, sem).wait()

        def pair_body(g, carry):
            b = g * 2
            process(b, 0)
            process(b + 1, 1)
            return carry

        lax.fori_loop(0, BPW // 2, pair_body, 0)
        wait_ctx(0)
        wait_neg(0)
        wait_ctx(1)
        wait_neg(1)
        pltpu.make_async_copy(negs_v0, negs_o.at[base], sem_o0).wait()
        pltpu.make_async_copy(negs_v1, negs_o.at[base], sem_o1).wait()
        pltpu.sync_copy(pos_v, pos_o.at[pl.ds(base, BPW)])

    return k(tgt, ctx, neg, i_emb, o_emb)


def _logsig(x):
    return jnp.minimum(x, 0.0) - jnp.log(1.0 + jnp.exp(-jnp.abs(x)))


def _tc_loss(pos_s, neg_s):
    def body(pos_ref, neg_ref, out_ref):
        p = jnp.sum(pos_ref[...], axis=1)
        lp = jnp.sum(_logsig(p))
        ln = jnp.sum(_logsig(-neg_ref[...]))
        out_ref[0, 0] = -(lp + ln)

    out = pl.pallas_call(
        body,
        out_shape=jax.ShapeDtypeStruct((1, 1), jnp.float32),
        in_specs=[pl.BlockSpec(memory_space=pltpu.VMEM),
                  pl.BlockSpec(memory_space=pltpu.VMEM)],
        out_specs=pl.BlockSpec(memory_space=pltpu.SMEM),
    )(pos_s, neg_s)
    return out[0, 0]


def kernel(target_wids, context_wids, i_embeddings, o_embeddings):
    tgt = target_wids.astype(jnp.int32)
    ctx = context_wids.astype(jnp.int32)
    neg = _neg_wids()
    pos_s, neg_s = _sc_scores(tgt, ctx, neg,
                              _pack_table(i_embeddings.astype(jnp.float32)),
                              _pack_table(o_embeddings.astype(jnp.float32)))
    return _tc_loss(pos_s, neg_s)


# speedup vs baseline: 1.4055x; 1.4055x over previous
import functools

import numpy as np
import jax
import jax.numpy as jnp
from jax import lax
from jax.experimental import pallas as pl
from jax.experimental.pallas import tpu as pltpu
from jax.experimental.pallas import tpu_sc as plsc

VOCAB = 100000
EMB = 128
N_NEG = 128
BATCH = 4096
CTX = 20

NC, NS = 2, 16
NW = NC * NS
BPW = BATCH // NW
EMBW = EMB // 2
NCH = EMBW // 16

_ILV = plsc.PackFormat.INTERLEAVED


def _neg_wids():
    wids = jax.random.randint(jax.random.key(1234), (BATCH, N_NEG), 0, VOCAB - 1)
    return wids.astype(jnp.int32)


def _pack_tables(i_emb, o_emb):
    cat = jnp.concatenate([o_emb.astype(jnp.bfloat16),
                           i_emb.astype(jnp.bfloat16)], axis=1)
    return lax.bitcast_convert_type(cat.reshape(VOCAB + 1, EMB, 2), jnp.int32)


def _sc_scores(tgt, ctx, neg, emb):
    mesh = plsc.VectorSubcoreMesh(
        core_axis_name="c", subcore_axis_name="s", num_cores=NC, num_subcores=NS)

    @functools.partial(
        pl.kernel,
        out_type=(jax.ShapeDtypeStruct((BATCH, 16), jnp.float32),
                  jax.ShapeDtypeStruct((BATCH, N_NEG), jnp.float32)),
        mesh=mesh,
        compiler_params=pltpu.CompilerParams(needs_layout_passes=False),
        scratch_types=[
            pltpu.VMEM((BPW, CTX), jnp.int32),
            pltpu.VMEM((BPW,), jnp.int32),
            pltpu.VMEM((BPW, N_NEG), jnp.int32),
            pltpu.VMEM((BPW, EMB), jnp.int32),
            pltpu.VMEM((CTX, EMB), jnp.int32),
            pltpu.VMEM((CTX, EMB), jnp.int32),
            pltpu.VMEM((N_NEG, EMB), jnp.int32),
            pltpu.VMEM((N_NEG, EMB), jnp.int32),
            pltpu.VMEM((BPW, 16), jnp.float32),
            pltpu.VMEM((N_NEG,), jnp.float32),
            pltpu.VMEM((N_NEG,), jnp.float32),
            pltpu.VMEM((N_NEG, 17), jnp.float32),
            pltpu.SemaphoreType.DMA,
            pltpu.SemaphoreType.DMA,
            pltpu.SemaphoreType.DMA,
            pltpu.SemaphoreType.DMA,
            pltpu.SemaphoreType.DMA,
            pltpu.SemaphoreType.DMA,
            pltpu.SemaphoreType.DMA,
        ],
    )
    def k(tgt_h, ctx_h, neg_h, emb_h, pos_o, negs_o,
          ctx_idx, tgt_idx, neg_idx, tgt_rows, ctx_rows0, ctx_rows1,
          neg_rows0, neg_rows1, pos_v, negs_v0, negs_v1, part_buf,
          sem, sem_c0, sem_c1, sem_n0, sem_n1, sem_o0, sem_o1):
        wid = lax.axis_index("s") * NC + lax.axis_index("c")
        base = wid * BPW
        pltpu.sync_copy(ctx_h.at[pl.ds(base, BPW)], ctx_idx)
        pltpu.sync_copy(tgt_h.at[pl.ds(base, BPW)], tgt_idx)
        pltpu.sync_copy(neg_h.at[pl.ds(base, BPW)], neg_idx)

        sems_c = (sem_c0, sem_c1)
        sems_n = (sem_n0, sem_n1)
        sems_o = (sem_o0, sem_o1)
        ctx_bufs = (ctx_rows0, ctx_rows1)
        neg_bufs = (neg_rows0, neg_rows1)
        out_bufs = (negs_v0, negs_v1)
        lane = lax.iota(jnp.int32, 16)

        def start_ctx(b, buf):
            bb = jnp.minimum(b, BPW - 1)
            pltpu.async_copy(emb_h.at[ctx_idx.at[bb]], ctx_bufs[buf], sems_c[buf])

        def start_neg(b, buf):
            bb = jnp.minimum(b, BPW - 1)
            pltpu.async_copy(emb_h.at[neg_idx.at[bb]], neg_bufs[buf], sems_n[buf])

        def wait_ctx(buf):
            pltpu.make_async_copy(
                emb_h.at[ctx_idx.at[0]], ctx_bufs[buf], sems_c[buf]).wait()

        def wait_neg(buf):
            pltpu.make_async_copy(
                emb_h.at[neg_idx.at[0]], neg_bufs[buf], sems_n[buf]).wait()

        def row_chunk_bf16(ref, r, j, half=0):
            return plsc.bitcast(
                ref[r, pl.ds(half * EMBW + j * 16, 16)], jnp.bfloat16)

        def process(b, buf):
            wait_ctx(buf)
            avg_bf = []
            for j in range(NCH):
                es, os_ = [], []
                for c in range(CTX):
                    e, o = plsc.unpack(row_chunk_bf16(ctx_bufs[buf], c, j, 1),
                                       format=_ILV)
                    es.append(e)
                    os_.append(o)
                for terms in (es, os_):
                    while len(terms) > 1:
                        nxt = [terms[i] + terms[i + 1]
                               for i in range(0, len(terms) - 1, 2)]
                        if len(terms) % 2:
                            nxt.append(terms[-1])
                        terms[:] = nxt
                avg_bf.append(plsc.pack(es[0] / jnp.float32(CTX),
                                        os_[0] / jnp.float32(CTX), format=_ILV))
            start_ctx(b + 2, buf)

            def row_dot_partial(ref, r):
                prods = [row_chunk_bf16(ref, r, j) * avg_bf[j]
                         for j in range(NCH)]
                q = (prods[0] + prods[1]) + (prods[2] + prods[3])
                e, o = plsc.unpack(q, format=_ILV)
                return e + o

            pos_v[b, :] = row_dot_partial(tgt_rows, b)

            wait_neg(buf)
            @pl.when(b >= 2)
            def _():
                pltpu.make_async_copy(
                    out_bufs[buf], negs_o.at[base], sems_o[buf]).wait()

            def neg_group(g):
                base_r = g * 16
                for r in range(16):
                    part_buf[base_r + r, pl.ds(0, 16)] = row_dot_partial(
                        neg_bufs[buf], base_r + r)
                row_idx = base_r + lane
                cols = [plsc.load_gather(
                            part_buf, [row_idx, jnp.full((16,), j, jnp.int32)])
                        for j in range(16)]
                while len(cols) > 1:
                    cols = [cols[i] + cols[i + 1] for i in range(0, len(cols), 2)]
                out_bufs[buf][pl.ds(base_r, 16)] = cols[0]

            plsc.parallel_loop(0, N_NEG // 16, 1)(neg_group)
            pltpu.async_copy(out_bufs[buf], negs_o.at[base + b], sems_o[buf])
            start_neg(b + 2, buf)

        start_ctx(jnp.int32(0), 0)
        start_neg(jnp.int32(0), 0)
        start_ctx(jnp.int32(1), 1)
        start_neg(jnp.int32(1), 1)
        pltpu.async_copy(emb_h.at[tgt_idx], tgt_rows, sem).wait()

        def pair_body(g, carry):
            b = g * 2
            process(b, 0)
            process(b + 1, 1)
            return carry

        lax.fori_loop(0, BPW // 2, pair_body, 0)
        wait_ctx(0)
        wait_neg(0)
        wait_ctx(1)
        wait_neg(1)
        pltpu.make_async_copy(negs_v0, negs_o.at[base], sem_o0).wait()
        pltpu.make_async_copy(negs_v1, negs_o.at[base], sem_o1).wait()
        pltpu.sync_copy(pos_v, pos_o.at[pl.ds(base, BPW)])

    return k(tgt, ctx, neg, emb)


def _logsig(x):
    return jnp.minimum(x, 0.0) - jnp.log(1.0 + jnp.exp(-jnp.abs(x)))


def _tc_loss(pos_s, neg_s):
    def body(pos_ref, neg_ref, out_ref):
        p = jnp.sum(pos_ref[...], axis=1)
        lp = jnp.sum(_logsig(p))
        ln = jnp.sum(_logsig(-neg_ref[...]))
        out_ref[0, 0] = -(lp + ln)

    out = pl.pallas_call(
        body,
        out_shape=jax.ShapeDtypeStruct((1, 1), jnp.float32),
        in_specs=[pl.BlockSpec(memory_space=pltpu.VMEM),
                  pl.BlockSpec(memory_space=pltpu.VMEM)],
        out_specs=pl.BlockSpec(memory_space=pltpu.SMEM),
    )(pos_s, neg_s)
    return out[0, 0]


def kernel(target_wids, context_wids, i_embeddings, o_embeddings):
    tgt = target_wids.astype(jnp.int32)
    ctx = context_wids.astype(jnp.int32)
    neg = _neg_wids()
    emb = _pack_tables(i_embeddings.astype(jnp.float32),
                       o_embeddings.astype(jnp.float32))
    pos_s, neg_s = _sc_scores(tgt, ctx, neg, emb)
    return _tc_loss(pos_s, neg_s)
